# baseline (speedup 1.0000x reference)
import jax
import jax.numpy as jnp
from jax import lax
from jax.experimental import pallas as pl
from jax.experimental.pallas import tpu as pltpu
from jax.experimental.pallas import tpu_sc as plsc

T = 2048
D = 1024
DFF = 768
E = 8
K = 2
BM = 256
NB = (T * K + (E - 1) * BM) // BM
NPAD = NB * BM
NC, NS = 2, 16
NW = NC * NS
TPW = T // NW
CH = 16
NCH = TPW // CH
LANES = 16
WL = 128


def _cumsum_lanes(x, n):
    s = 1
    while s < n:
        shifted = jnp.concatenate(
            [jnp.zeros((x.shape[0], s), x.dtype), x[:, :-s]], axis=1)
        x = x + shifted
        s *= 2
    return x


def _meta_body(ids_ref, w_ref, pe_ref, po_ref, be_ref, nb_ref, wr0_ref, wr1_ref):
    ids = ids_ref[...]
    e0 = ids[:, 0][None, :]
    e1 = ids[:, 1][None, :]
    lanes_e = jax.lax.broadcasted_iota(jnp.int32, (E, T), 0)
    oh0 = (lanes_e == e0)
    oh1 = (lanes_e == e1)
    rowcnt = oh0.astype(jnp.int32) + oh1.astype(jnp.int32)
    incl = _cumsum_lanes(rowcnt, T)
    excl = incl - rowcnt
    rank0 = jnp.sum(jnp.where(oh0, excl, 0), axis=0)
    rank1 = jnp.sum(jnp.where(oh1, excl, 0), axis=0) + (ids[:, 0] == ids[:, 1])
    counts = incl[:, T - 1:T]
    padded = ((counts + BM - 1) // BM) * BM
    cum = padded
    s = 1
    while s < E:
        cum = cum + jnp.concatenate(
            [jnp.zeros((s, 1), jnp.int32), cum[:-s, :]], axis=0)
        s *= 2
    offs = cum - padded
    off0 = jnp.sum(jnp.where(oh0, offs, 0), axis=0)
    off1 = jnp.sum(jnp.where(oh1, offs, 0), axis=0)
    pe_ref[...] = (off0 + rank0)[None, :]
    po_ref[...] = (off1 + rank1)[None, :]
    bstart = jax.lax.broadcasted_iota(jnp.int32, (E, NB), 1) * BM
    be = jnp.sum((cum <= bstart).astype(jnp.int32), axis=0)
    be_ref[...] = jnp.minimum(be, E - 1)[None, :]
    nb_ref[...] = cum[E - 1:E, :] // BM
    w = w_ref[...]
    wr0_ref[...] = jnp.broadcast_to(w[:, :1], (T, WL))
    wr1_ref[...] = jnp.broadcast_to(w[:, 1:2], (T, WL))


def _meta(ids, topk_weights):
    pe, po, be, nb, wr0, wr1 = pl.pallas_call(
        _meta_body,
        out_shape=(jax.ShapeDtypeStruct((1, T), jnp.int32),
                   jax.ShapeDtypeStruct((1, T), jnp.int32),
                   jax.ShapeDtypeStruct((1, NB), jnp.int32),
                   jax.ShapeDtypeStruct((1, 1), jnp.int32),
                   jax.ShapeDtypeStruct((T, WL), jnp.float32),
                   jax.ShapeDtypeStruct((T, WL), jnp.float32)),
    )(ids, topk_weights)
    return pe.reshape(T), po.reshape(T), be.reshape(NB), nb.reshape(1), wr0, wr1


def _mesh():
    return plsc.VectorSubcoreMesh(core_axis_name="c", subcore_axis_name="s",
                                  num_cores=NC, num_subcores=NS)


def _dispatch_body(x_hbm, pe_hbm, po_hbm, wr0_hbm, wr1_hbm, xs_hbm, pw_hbm,
                   xbuf, idxe, idxo, wb0, wb1, sem_ld, sem_st):
    wid = lax.axis_index("s") * NC + lax.axis_index("c")
    tb = wid * TPW
    sl = pl.ds(tb, TPW)
    loads = [
        pltpu.async_copy(x_hbm.at[sl], xbuf, sem_ld),
        pltpu.async_copy(pe_hbm.at[sl], idxe, sem_ld),
        pltpu.async_copy(po_hbm.at[sl], idxo, sem_ld),
        pltpu.async_copy(wr0_hbm.at[sl], wb0, sem_ld),
        pltpu.async_copy(wr1_hbm.at[sl], wb1, sem_ld),
    ]
    for cp in loads:
        cp.wait()
    stores = [
        pltpu.async_copy(xbuf, xs_hbm.at[idxe], sem_st),
        pltpu.async_copy(xbuf, xs_hbm.at[idxo], sem_st),
        pltpu.async_copy(wb0, pw_hbm.at[idxe], sem_st),
        pltpu.async_copy(wb1, pw_hbm.at[idxo], sem_st),
    ]
    for cp in stores:
        cp.wait()


def _dispatch(hidden_states, pos_e, pos_o, wr0, wr1):
    return pl.kernel(
        _dispatch_body,
        out_type=(jax.ShapeDtypeStruct((NPAD, D), jnp.float32),
                  jax.ShapeDtypeStruct((NPAD, WL), jnp.float32)),
        mesh=_mesh(),
        scratch_types=[
            pltpu.VMEM((TPW, D), jnp.float32),
            pltpu.VMEM((TPW,), jnp.int32),
            pltpu.VMEM((TPW,), jnp.int32),
            pltpu.VMEM((TPW, WL), jnp.float32),
            pltpu.VMEM((TPW, WL), jnp.float32),
            pltpu.SemaphoreType.DMA,
            pltpu.SemaphoreType.DMA,
        ],
    )(hidden_states, pos_e, pos_o, wr0, wr1)


def _combine_body(y_hbm, pe_hbm, po_hbm, out_hbm,
                  idxe, idxo, b0a, b1a, b0b, b1b, sema, semb, sem_st):
    wid = lax.axis_index("s") * NC + lax.axis_index("c")
    tb = wid * TPW
    pltpu.sync_copy(pe_hbm.at[pl.ds(wid * NCH, NCH)], idxe)
    pltpu.sync_copy(po_hbm.at[pl.ds(wid * NCH, NCH)], idxo)
    bufs = [(b0a, b1a, sema), (b0b, b1b, semb)]

    def start(c):
        b0, b1, sem = bufs[c % 2]
        return (pltpu.async_copy(y_hbm.at[idxe.at[c]], b0, sem),
                pltpu.async_copy(y_hbm.at[idxo.at[c]], b1, sem))

    pending = start(0)
    st_prev = [None, None]
    for c in range(NCH):
        nxt = None
        if c + 1 < NCH:
            if st_prev[(c + 1) % 2] is not None:
                st_prev[(c + 1) % 2].wait()
                st_prev[(c + 1) % 2] = None
            nxt = start(c + 1)
        for cp in pending:
            cp.wait()
        b0, b1, _ = bufs[c % 2]

        def row_add(j, carry):
            for s in range(D // LANES):
                dsl = pl.ds(s * LANES, LANES)
                b0[j, dsl] = b0[j, dsl] + b1[j, dsl]
            return carry

        lax.fori_loop(0, CH, row_add, 0)
        st_prev[c % 2] = pltpu.async_copy(
            b0, out_hbm.at[pl.ds(tb + c * CH, CH)], sem_st)
        pending = nxt
    for st in st_prev:
        if st is not None:
            st.wait()


def _combine(y_sorted, pos_e, pos_o):
    return pl.kernel(
        _combine_body,
        out_type=jax.ShapeDtypeStruct((T, D), jnp.float32),
        mesh=_mesh(),
        scratch_types=[
            pltpu.VMEM((NCH, CH), jnp.int32),
            pltpu.VMEM((NCH, CH), jnp.int32),
            pltpu.VMEM((CH, D), jnp.float32),
            pltpu.VMEM((CH, D), jnp.float32),
            pltpu.VMEM((CH, D), jnp.float32),
            pltpu.VMEM((CH, D), jnp.float32),
            pltpu.SemaphoreType.DMA,
            pltpu.SemaphoreType.DMA,
            pltpu.SemaphoreType.DMA,
        ],
    )(y_sorted, pos_e.reshape(T // CH, CH), pos_o.reshape(T // CH, CH))


def _gemm_body(be_ref, nb_ref, x_ref, w1_ref, w2_ref, pw_ref, o_ref, gug_ref):
    i = pl.program_id(0)
    j = pl.program_id(1)

    @pl.when(i < nb_ref[0])
    def _():
        x = x_ref[...]
        half = lax.dot_general(x, w1_ref[0], (((1,), (1,)), ((), ())),
                               preferred_element_type=jnp.float32)

        @pl.when(j == 0)
        def _():
            gug_ref[...] = half

        @pl.when(j == 1)
        def _():
            gate = gug_ref[...]
            act = gate * lax.logistic(gate) * half
            y = lax.dot_general(act, w2_ref[0], (((1,), (1,)), ((), ())),
                                preferred_element_type=jnp.float32)
            o_ref[...] = y * pw_ref[:, :1]


def _gemm(block_expert, nblk, x_sorted, w1, w2, pw2d):
    grid_spec = pltpu.PrefetchScalarGridSpec(
        num_scalar_prefetch=2,
        grid=(NB, 2),
        in_specs=[
            pl.BlockSpec((BM, D),
                         lambda i, j, be, nb: (jnp.minimum(i, nb[0] - 1), 0)),
            pl.BlockSpec((1, DFF, D), lambda i, j, be, nb: (be[i], j, 0)),
            pl.BlockSpec((1, D, DFF), lambda i, j, be, nb: (be[i], 0, 0)),
            pl.BlockSpec((BM, WL),
                         lambda i, j, be, nb: (jnp.minimum(i, nb[0] - 1), 0)),
        ],
        out_specs=pl.BlockSpec((BM, D),
                               lambda i, j, be, nb: (jnp.minimum(i, nb[0] - 1), 0)),
        scratch_shapes=[pltpu.VMEM((BM, DFF), jnp.float32)],
    )
    return pl.pallas_call(
        _gemm_body,
        grid_spec=grid_spec,
        out_shape=jax.ShapeDtypeStruct((NPAD, D), jnp.float32),
    )(block_expert, nblk, x_sorted, w1, w2, pw2d)


def kernel(hidden_states, topk_weights, topk_ids, w1, w2):
    ids = topk_ids.astype(jnp.int32)
    pos_e, pos_o, block_expert, nblk, wr0, wr1 = _meta(ids, topk_weights)
    x_sorted, pw2d = _dispatch(hidden_states, pos_e, pos_o, wr0, wr1)
    y_sorted = _gemm(block_expert, nblk, x_sorted, w1, w2, pw2d)
    return _combine(y_sorted, pos_e, pos_o)


# speedup vs baseline: 1.2190x; 1.2190x over previous
import jax
import jax.numpy as jnp
from jax import lax
from jax.experimental import pallas as pl
from jax.experimental.pallas import tpu as pltpu
from jax.experimental.pallas import tpu_sc as plsc

T = 2048
D = 1024
DFF = 768
E = 8
K = 2
BM = 256
NB = (T * K + (E - 1) * BM) // BM
NPAD = NB * BM
NC, NS = 2, 16
NW = NC * NS
TPW = T // NW
CH = 16
NCH = TPW // CH
LANES = 16
WL = 128


def _cumsum_lanes(x, n):
    s = 1
    while s < n:
        shifted = jnp.concatenate(
            [jnp.zeros((x.shape[0], s), x.dtype), x[:, :-s]], axis=1)
        x = x + shifted
        s *= 2
    return x


def _meta_body(ids_ref, w_ref, pe_ref, po_ref, be_ref, nb_ref, wr0_ref, wr1_ref):
    ids = ids_ref[...]
    e0 = ids[:, 0][None, :]
    e1 = ids[:, 1][None, :]
    lanes_e = jax.lax.broadcasted_iota(jnp.int32, (E, T), 0)
    oh0 = (lanes_e == e0)
    oh1 = (lanes_e == e1)
    rowcnt = oh0.astype(jnp.int32) + oh1.astype(jnp.int32)
    incl = _cumsum_lanes(rowcnt, T)
    excl = incl - rowcnt
    rank0 = jnp.sum(jnp.where(oh0, excl, 0), axis=0)
    rank1 = jnp.sum(jnp.where(oh1, excl, 0), axis=0) + (ids[:, 0] == ids[:, 1])
    counts = incl[:, T - 1:T]
    padded = ((counts + BM - 1) // BM) * BM
    cum = padded
    s = 1
    while s < E:
        cum = cum + jnp.concatenate(
            [jnp.zeros((s, 1), jnp.int32), cum[:-s, :]], axis=0)
        s *= 2
    offs = cum - padded
    off0 = jnp.sum(jnp.where(oh0, offs, 0), axis=0)
    off1 = jnp.sum(jnp.where(oh1, offs, 0), axis=0)
    pe_ref[...] = (off0 + rank0)[None, :]
    po_ref[...] = (off1 + rank1)[None, :]
    bstart = jax.lax.broadcasted_iota(jnp.int32, (E, NB), 1) * BM
    be = jnp.sum((cum <= bstart).astype(jnp.int32), axis=0)
    be_ref[...] = jnp.minimum(be, E - 1)[None, :]
    nb_ref[...] = cum[E - 1:E, :] // BM
    w = w_ref[...]
    wr0_ref[...] = jnp.broadcast_to(w[:, :1], (T, WL))
    wr1_ref[...] = jnp.broadcast_to(w[:, 1:2], (T, WL))


def _meta(ids, topk_weights):
    pe, po, be, nb, wr0, wr1 = pl.pallas_call(
        _meta_body,
        out_shape=(jax.ShapeDtypeStruct((1, T), jnp.int32),
                   jax.ShapeDtypeStruct((1, T), jnp.int32),
                   jax.ShapeDtypeStruct((1, NB), jnp.int32),
                   jax.ShapeDtypeStruct((1, 1), jnp.int32),
                   jax.ShapeDtypeStruct((T, WL), jnp.float32),
                   jax.ShapeDtypeStruct((T, WL), jnp.float32)),
    )(ids, topk_weights)
    return pe.reshape(T), po.reshape(T), be.reshape(NB), nb.reshape(1), wr0, wr1


def _mesh():
    return plsc.VectorSubcoreMesh(core_axis_name="c", subcore_axis_name="s",
                                  num_cores=NC, num_subcores=NS)


def _dispatch_body(x_hbm, pe_hbm, po_hbm, wr0_hbm, wr1_hbm, xs_hbm, pw_hbm,
                   xbuf, idxe, idxo, wb0, wb1, sem_ld, sem_st):
    wid = lax.axis_index("s") * NC + lax.axis_index("c")
    tb = wid * TPW
    sl = pl.ds(tb, TPW)
    loads = [
        pltpu.async_copy(x_hbm.at[sl], xbuf, sem_ld),
        pltpu.async_copy(pe_hbm.at[sl], idxe, sem_ld),
        pltpu.async_copy(po_hbm.at[sl], idxo, sem_ld),
        pltpu.async_copy(wr0_hbm.at[sl], wb0, sem_ld),
        pltpu.async_copy(wr1_hbm.at[sl], wb1, sem_ld),
    ]
    for cp in loads:
        cp.wait()
    stores = [
        pltpu.async_copy(xbuf, xs_hbm.at[idxe], sem_st),
        pltpu.async_copy(xbuf, xs_hbm.at[idxo], sem_st),
        pltpu.async_copy(wb0, pw_hbm.at[idxe], sem_st),
        pltpu.async_copy(wb1, pw_hbm.at[idxo], sem_st),
    ]
    for cp in stores:
        cp.wait()


def _dispatch(hidden_states, pos_e, pos_o, wr0, wr1):
    return pl.kernel(
        _dispatch_body,
        out_type=(jax.ShapeDtypeStruct((NPAD, D), jnp.float32),
                  jax.ShapeDtypeStruct((NPAD, WL), jnp.float32)),
        mesh=_mesh(),
        scratch_types=[
            pltpu.VMEM((TPW, D), jnp.float32),
            pltpu.VMEM((TPW,), jnp.int32),
            pltpu.VMEM((TPW,), jnp.int32),
            pltpu.VMEM((TPW, WL), jnp.float32),
            pltpu.VMEM((TPW, WL), jnp.float32),
            pltpu.SemaphoreType.DMA,
            pltpu.SemaphoreType.DMA,
        ],
    )(hidden_states, pos_e, pos_o, wr0, wr1)


def _combine_body(y_hbm, pe_hbm, po_hbm, out_hbm,
                  idxe, idxo, b0a, b1a, b0b, b1b, sema, semb, sem_st):
    wid = lax.axis_index("s") * NC + lax.axis_index("c")
    tb = wid * TPW
    pltpu.sync_copy(pe_hbm.at[pl.ds(wid * NCH, NCH)], idxe)
    pltpu.sync_copy(po_hbm.at[pl.ds(wid * NCH, NCH)], idxo)
    bufs = [(b0a, b1a, sema), (b0b, b1b, semb)]

    def start(c):
        b0, b1, sem = bufs[c % 2]
        return (pltpu.async_copy(y_hbm.at[idxe.at[c]], b0, sem),
                pltpu.async_copy(y_hbm.at[idxo.at[c]], b1, sem))

    pending = start(0)
    st_prev = [None, None]
    for c in range(NCH):
        nxt = None
        if c + 1 < NCH:
            if st_prev[(c + 1) % 2] is not None:
                st_prev[(c + 1) % 2].wait()
                st_prev[(c + 1) % 2] = None
            nxt = start(c + 1)
        for cp in pending:
            cp.wait()
        b0, b1, _ = bufs[c % 2]

        def row_add(j, carry):
            for s in range(D // LANES):
                dsl = pl.ds(s * LANES, LANES)
                b0[j, dsl] = b0[j, dsl] + b1[j, dsl]
            return carry

        lax.fori_loop(0, CH, row_add, 0)
        st_prev[c % 2] = pltpu.async_copy(
            b0, out_hbm.at[pl.ds(tb + c * CH, CH)], sem_st)
        pending = nxt
    for st in st_prev:
        if st is not None:
            st.wait()


def _combine(y_sorted, pos_e, pos_o):
    return pl.kernel(
        _combine_body,
        out_type=jax.ShapeDtypeStruct((T, D), jnp.float32),
        mesh=_mesh(),
        scratch_types=[
            pltpu.VMEM((NCH, CH), jnp.int32),
            pltpu.VMEM((NCH, CH), jnp.int32),
            pltpu.VMEM((CH, D), jnp.float32),
            pltpu.VMEM((CH, D), jnp.float32),
            pltpu.VMEM((CH, D), jnp.float32),
            pltpu.VMEM((CH, D), jnp.float32),
            pltpu.SemaphoreType.DMA,
            pltpu.SemaphoreType.DMA,
            pltpu.SemaphoreType.DMA,
        ],
    )(y_sorted, pos_e.reshape(T // CH, CH), pos_o.reshape(T // CH, CH))


def _gemm_body(be_ref, nb_ref, x_ref, w1_ref, w2_ref, pw_ref, o_ref):
    @pl.when(pl.program_id(0) < nb_ref[0])
    def _():
        x = x_ref[...]
        gu = lax.dot_general(x, w1_ref[0], (((1,), (1,)), ((), ())),
                             preferred_element_type=jnp.float32)
        gate = gu[:, :DFF]
        up = gu[:, DFF:]
        act = gate * lax.logistic(gate) * up
        y = lax.dot_general(act, w2_ref[0], (((1,), (1,)), ((), ())),
                            preferred_element_type=jnp.float32)
        o_ref[...] = y * pw_ref[:, :1]


def _gemm(block_expert, nblk, x_sorted, w1, w2, pw2d):
    grid_spec = pltpu.PrefetchScalarGridSpec(
        num_scalar_prefetch=2,
        grid=(NB,),
        in_specs=[
            pl.BlockSpec((BM, D),
                         lambda i, be, nb: (jnp.minimum(i, nb[0] - 1), 0)),
            pl.BlockSpec((1, 2 * DFF, D), lambda i, be, nb: (be[i], 0, 0)),
            pl.BlockSpec((1, D, DFF), lambda i, be, nb: (be[i], 0, 0)),
            pl.BlockSpec((BM, WL),
                         lambda i, be, nb: (jnp.minimum(i, nb[0] - 1), 0)),
        ],
        out_specs=pl.BlockSpec((BM, D),
                               lambda i, be, nb: (jnp.minimum(i, nb[0] - 1), 0)),
    )
    return pl.pallas_call(
        _gemm_body,
        grid_spec=grid_spec,
        out_shape=jax.ShapeDtypeStruct((NPAD, D), jnp.float32),
    )(block_expert, nblk, x_sorted, w1, w2, pw2d)


def kernel(hidden_states, topk_weights, topk_ids, w1, w2):
    ids = topk_ids.astype(jnp.int32)
    pos_e, pos_o, block_expert, nblk, wr0, wr1 = _meta(ids, topk_weights)
    x_sorted, pw2d = _dispatch(hidden_states, pos_e, pos_o, wr0, wr1)
    y_sorted = _gemm(block_expert, nblk, x_sorted, w1, w2, pw2d)
    return _combine(y_sorted, pos_e, pos_o)
